# Initial kernel scaffold; baseline (speedup 1.0000x reference)
import jax
import jax.numpy as jnp
from jax.experimental import pallas as pl


def kernel(h, edge_index, W1, b1, Wg1, bg1, Wg2, bg2):
    raise NotImplementedError("write your pallas kernel here")


# speedup vs baseline: 8.7739x; 8.7739x over previous
import functools

import jax
import jax.numpy as jnp
from jax import lax
from jax.experimental import pallas as pl
from jax.experimental.pallas import tpu as pltpu
from jax.experimental.pallas import tpu_sc as plsc

NC = 2
NS = 16
NW = NC * NS
L = 16
C = 80


def _sc_mesh():
    return plsc.VectorSubcoreMesh(
        core_axis_name="c", subcore_axis_name="s", num_cores=NC,
        num_subcores=NS)


def _deg_body(nchunk, npt, row_hbm, ones_hbm, zero_hbm, out_hbm,
              idxc, ones_v, acc, sem):
    cid = lax.axis_index("c")
    sid = lax.axis_index("s")
    wid = sid * NC + cid
    pltpu.sync_copy(zero_hbm, acc.at[pl.ds(sid * npt, npt)])
    pltpu.sync_copy(ones_hbm, ones_v)
    plsc.subcore_barrier()

    def chunk(j, carry):
        pltpu.sync_copy(row_hbm.at[wid, j], idxc)
        pltpu.sync_copy(ones_v, acc.at[idxc], add=True)
        return carry

    lax.fori_loop(0, nchunk, chunk, 0)
    plsc.subcore_barrier()
    pltpu.sync_copy(acc.at[pl.ds(sid * npt, npt)], out_hbm.at[cid, sid])


def _deg_kernel(n, e, row3):
    nchunk = row3.shape[1]
    npt = n // NS
    ones = jnp.concatenate(
        [jnp.ones((C, 1), jnp.float32), jnp.zeros((C, 7), jnp.float32)], 1)
    zero = jnp.zeros((npt, 8), jnp.float32)
    body = functools.partial(_deg_body, nchunk, npt)
    fn = pl.kernel(
        body,
        out_type=jax.ShapeDtypeStruct((NC, NS, n // NS, 8), jnp.float32),
        mesh=_sc_mesh(),
        compiler_params=pltpu.CompilerParams(needs_layout_passes=False),
        scratch_types=[
            pltpu.VMEM((C,), jnp.int32),
            pltpu.VMEM((C, 8), jnp.float32),
            pltpu.VMEM_SHARED((n, 8), jnp.float32),
            pltpu.SemaphoreType.DMA,
        ],
    )
    return fn(row3, ones, zero)


def _layer_body(n, nchunk, npt, h_hbm, row_hbm, col_hbm, a_hbm, b_hbm,
                nd_hbm, zero_hbm, out_hbm,
                idxg, idxs, a_l, b_l, nd_l, wch, rows, acc,
                sem):
    cid = lax.axis_index("c")
    sid = lax.axis_index("s")
    wid = sid * NC + cid
    pltpu.sync_copy(zero_hbm, acc.at[pl.ds(sid * npt, npt)])
    pltpu.sync_copy(a_hbm, a_l)
    pltpu.sync_copy(b_hbm, b_l)
    pltpu.sync_copy(nd_hbm, nd_l)
    plsc.subcore_barrier()

    def chunk(j, carry):
        pltpu.sync_copy(row_hbm.at[wid, j], idxg)
        pltpu.sync_copy(col_hbm.at[wid, j], idxs)
        pltpu.async_copy(h_hbm.at[idxg], rows, sem).wait()
        for k in range(C // L):
            ir = idxg[pl.ds(k * L, L)]
            ic = idxs[pl.ds(k * L, L)]
            av = plsc.load_gather(a_l, [ir])
            bv = plsc.load_gather(b_l, [ic])
            nr = plsc.load_gather(nd_l, [ir])
            ncv = plsc.load_gather(nd_l, [ic])
            x = av + bv
            t = jnp.exp(-2.0 * jnp.abs(x))
            g = jnp.sign(x) * (1.0 - t) / (1.0 + t)
            wch[pl.ds(k * L, L)] = g * nr * ncv

        def scale(ei, carry2):
            w = plsc.load_gather(wch, [jnp.full((L,), ei, jnp.int32)])
            for v in range(8):
                rows[ei, pl.ds(v * L, L)] = rows[ei, pl.ds(v * L, L)] * w
            return carry2

        lax.fori_loop(0, C, scale, 0)
        pltpu.sync_copy(rows, acc.at[idxs], add=True)
        return carry

    lax.fori_loop(0, nchunk, chunk, 0)
    plsc.subcore_barrier()
    pltpu.sync_copy(acc.at[pl.ds(sid * npt, npt)], out_hbm.at[cid, sid])


def _layer_kernel(h_src, row3, col3, a, b, nd):
    n, d = h_src.shape
    nchunk = row3.shape[1]
    npt = n // NS
    zero = jnp.zeros((npt, d), jnp.float32)
    body = functools.partial(_layer_body, n, nchunk, npt)
    fn = pl.kernel(
        body,
        out_type=jax.ShapeDtypeStruct((NC, NS, n // NS, d), jnp.float32),
        mesh=_sc_mesh(),
        compiler_params=pltpu.CompilerParams(needs_layout_passes=False),
        scratch_types=[
            pltpu.VMEM((C,), jnp.int32),
            pltpu.VMEM((C,), jnp.int32),
            pltpu.VMEM((n,), jnp.float32),
            pltpu.VMEM((n,), jnp.float32),
            pltpu.VMEM((n,), jnp.float32),
            pltpu.VMEM((C,), jnp.float32),
            pltpu.VMEM((C, d), jnp.float32),
            pltpu.VMEM_SHARED((n, d), jnp.float32),
            pltpu.SemaphoreType.DMA,
        ],
    )
    return fn(h_src, row3, col3, a, b, nd, zero)


def _tc_pre_body(h_ref, w1_ref, b1_ref, wab_ref, bg_ref, degp_ref,
                 h1_ref, a_ref, b_ref, nd_ref):
    h1 = lax.dot_general(h_ref[...], w1_ref[...],
                         (((1,), (1,)), ((), ())),
                         preferred_element_type=jnp.float32)
    h1 = jnp.maximum(h1 + b1_ref[...][None, :], 0.0)
    h1_ref[...] = h1
    ab = lax.dot_general(h1, wab_ref[...], (((1,), (0,)), ((), ())),
                         preferred_element_type=jnp.float32)
    a_ref[...] = ab[:, 0:1] + bg_ref[0, 0]
    b_ref[...] = ab[:, 1:2]
    deg = degp_ref[0, :, 0:1] + degp_ref[1, :, 0:1]
    nd_ref[...] = lax.rsqrt(jnp.maximum(deg, 1.0))


def _tc_pre(h, w1, b1, wab, bg, degp):
    n, d = h.shape
    return pl.pallas_call(
        _tc_pre_body,
        out_shape=[
            jax.ShapeDtypeStruct((n, d), jnp.float32),
            jax.ShapeDtypeStruct((n, 1), jnp.float32),
            jax.ShapeDtypeStruct((n, 1), jnp.float32),
            jax.ShapeDtypeStruct((n, 1), jnp.float32),
        ],
    )(h, w1, b1, wab, bg, degp)


def _tc_mid_body(h1_ref, part_ref, wab_ref, bg_ref, h2_ref, a_ref, b_ref):
    h2 = 0.3 * h1_ref[...] + part_ref[0] + part_ref[1]
    h2_ref[...] = h2
    ab = lax.dot_general(h2, wab_ref[...], (((1,), (0,)), ((), ())),
                         preferred_element_type=jnp.float32)
    a_ref[...] = ab[:, 0:1] + bg_ref[0, 0]
    b_ref[...] = ab[:, 1:2]


def _tc_mid(h1, part, wab, bg):
    n, d = h1.shape
    return pl.pallas_call(
        _tc_mid_body,
        out_shape=[
            jax.ShapeDtypeStruct((n, d), jnp.float32),
            jax.ShapeDtypeStruct((n, 1), jnp.float32),
            jax.ShapeDtypeStruct((n, 1), jnp.float32),
        ],
    )(h1, part, wab, bg)


def _tc_fin_body(h1_ref, part_ref, out_ref):
    out_ref[...] = 0.3 * h1_ref[...] + part_ref[0] + part_ref[1]


def _tc_fin(h1, part):
    n, d = h1.shape
    return pl.pallas_call(
        _tc_fin_body,
        out_shape=jax.ShapeDtypeStruct((n, d), jnp.float32),
    )(h1, part)


def kernel(h, edge_index, W1, b1, Wg1, bg1, Wg2, bg2):
    n, d = h.shape
    e = edge_index.shape[1]
    epw = e // NW
    nchunk = epw // C
    row3 = edge_index[0].reshape(NW, nchunk, C)
    col3 = edge_index[1].reshape(NW, nchunk, C)
    wab1 = jnp.stack([Wg1[0, :d], Wg1[0, d:]], axis=1)
    wab2 = jnp.stack([Wg2[0, :d], Wg2[0, d:]], axis=1)
    bg1m = bg1.reshape(1, 1)
    bg2m = bg2.reshape(1, 1)

    degp = _deg_kernel(n, e, row3).reshape(NC, n, 8)
    h1, a1, b1v, nd = _tc_pre(h, W1, b1, wab1, bg1m, degp)
    a1 = a1.reshape(n)
    b1v = b1v.reshape(n)
    nd = nd.reshape(n)

    part1 = _layer_kernel(h1, row3, col3, a1, b1v, nd).reshape(NC, n, d)
    h2, a2, b2v = _tc_mid(h1, part1, wab2, bg2m)
    a2 = a2.reshape(n)
    b2v = b2v.reshape(n)

    part2 = _layer_kernel(h2, row3, col3, a2, b2v, nd).reshape(NC, n, d)
    return _tc_fin(h1, part2)
